# baseline (speedup 1.0000x reference)
import functools

import jax
import jax.numpy as jnp
from jax import lax
from jax.experimental import pallas as pl
from jax.experimental.pallas import tpu as pltpu
from jax.experimental.pallas import tpu_sc as plsc

NC = 2
NS = 16
N_PAD = 10240
CHUNK = 128
DH = 128
DEGW = 16
NUM_G = 64
BM = 1280


def _mesh():
    return plsc.VectorSubcoreMesh(core_axis_name="c", subcore_axis_name="s")


DEG_GRP = 8


def _deg_body(dst_hbm, out_hbm, dst_all, ones_v, acc, sem):
    c = lax.axis_index("c")
    s = lax.axis_index("s")
    wid = c * NS + s
    n_chunks = dst_hbm.shape[1]
    rows_per_tile = N_PAD // NS

    def fill(i, _):
        ones_v[i, :] = jnp.full((DEGW,), 1.0, jnp.float32)
        ones_v[CHUNK + i, :] = jnp.zeros((DEGW,), jnp.float32)
        return 0

    lax.fori_loop(0, CHUNK, fill, 0)

    pltpu.sync_copy(dst_hbm.at[wid], dst_all)

    def zero(j, _):
        pltpu.sync_copy(ones_v.at[pl.ds(CHUNK, CHUNK)],
                        acc.at[pl.ds(s * rows_per_tile + j * CHUNK, CHUNK)])
        return 0

    lax.fori_loop(0, rows_per_tile // CHUNK, zero, 0)
    plsc.subcore_barrier()

    ones = ones_v.at[pl.ds(0, CHUNK)]

    def group(g, _):
        for b in range(DEG_GRP):
            j = g * DEG_GRP + b
            pltpu.async_copy(ones, acc.at[dst_all.at[j]], sem, add=True)
        for b in range(DEG_GRP):
            pltpu.make_async_copy(ones, acc.at[dst_all.at[0]], sem).wait()
        return 0

    lax.fori_loop(0, n_chunks // DEG_GRP, group, 0)
    plsc.subcore_barrier()

    pltpu.sync_copy(acc.at[pl.ds(s * rows_per_tile, rows_per_tile)],
                    out_hbm.at[c, pl.ds(s * rows_per_tile, rows_per_tile)])


def _deg_call(dst3):
    n_chunks = dst3.shape[1]
    return pl.kernel(
        _deg_body,
        out_type=jax.ShapeDtypeStruct((NC, N_PAD, DEGW), jnp.float32),
        mesh=_mesh(),
        scratch_types=[
            pltpu.VMEM((n_chunks, CHUNK), jnp.int32),
            pltpu.VMEM((2 * CHUNK, DEGW), jnp.float32),
            pltpu.VMEM_SHARED((N_PAD, DEGW), jnp.float32),
            pltpu.SemaphoreType.DMA,
        ],
    )(dst3)


def _edge_body(src_hbm, dst_hbm, u_hbm, out_hbm, src_v, dst_v, rows,
               acc, isem, gsem, ssem):
    c = lax.axis_index("c")
    s = lax.axis_index("s")
    wid = c * NS + s
    ep = src_hbm.shape[0]
    edges_per_tile = ep // (NC * NS)
    n_chunks = edges_per_tile // CHUNK
    rows_per_tile = N_PAD // NS

    def fill(i, _):
        def fill_lane(k, _):
            rows[0, i, pl.ds(k * 16, 16)] = jnp.zeros((16,), jnp.float32)
            return 0
        lax.fori_loop(0, DH // 16, fill_lane, 0)
        return 0

    lax.fori_loop(0, CHUNK, fill, 0)

    def zero(j, _):
        pltpu.sync_copy(rows.at[0],
                        acc.at[pl.ds(s * rows_per_tile + j * CHUNK, CHUNK)])
        return 0

    lax.fori_loop(0, rows_per_tile // CHUNK, zero, 0)
    plsc.subcore_barrier()

    def idx_load(j, slot):
        base = pl.multiple_of(wid * edges_per_tile + j * CHUNK, CHUNK)
        pltpu.async_copy(src_hbm.at[pl.ds(base, CHUNK)], src_v.at[slot],
                         isem)
        pltpu.async_copy(dst_hbm.at[pl.ds(base, CHUNK)], dst_v.at[slot],
                         isem)

    def idx_wait():
        pltpu.make_async_copy(src_hbm.at[pl.ds(0, CHUNK)], src_v.at[0],
                              isem).wait()
        pltpu.make_async_copy(dst_hbm.at[pl.ds(0, CHUNK)], dst_v.at[0],
                              isem).wait()

    def gather(buf, slot):
        pltpu.async_copy(u_hbm.at[src_v.at[slot]], rows.at[buf], gsem)

    def gather_wait(buf):
        pltpu.make_async_copy(u_hbm.at[src_v.at[0]], rows.at[buf],
                              gsem).wait()

    def scatter(buf, slot):
        pltpu.async_copy(rows.at[buf], acc.at[dst_v.at[slot]], ssem,
                         add=True)

    def scatter_wait(buf):
        pltpu.make_async_copy(rows.at[buf], acc.at[dst_v.at[0]],
                              ssem).wait()

    idx_load(0, 0)
    idx_load(1, 1)
    idx_wait()
    gather(0, 0)

    def step(j, _):
        p = j % 2
        q = 1 - p
        gather_wait(p)
        scatter(p, j % 3)

        @pl.when(j + 1 < n_chunks)
        def _():
            @pl.when(j >= 1)
            def _():
                scatter_wait(q)

            @pl.when(j + 2 < n_chunks)
            def _():
                idx_load(j + 2, (j + 2) % 3)

            idx_wait()
            gather(q, (j + 1) % 3)
        return 0

    lax.fori_loop(0, n_chunks, step, 0)
    scatter_wait((n_chunks - 1) % 2)
    plsc.subcore_barrier()

    pltpu.sync_copy(acc.at[pl.ds(s * rows_per_tile, rows_per_tile)],
                    out_hbm.at[c, pl.ds(s * rows_per_tile, rows_per_tile)])


def _edge_call(src3, dst3, u):
    return pl.kernel(
        _edge_body,
        out_type=jax.ShapeDtypeStruct((NC, N_PAD, DH), jnp.float32),
        mesh=_mesh(),
        scratch_types=[
            pltpu.VMEM((3, CHUNK), jnp.int32),
            pltpu.VMEM((3, CHUNK), jnp.int32),
            pltpu.VMEM((2, CHUNK, DH), jnp.float32),
            pltpu.VMEM_SHARED((N_PAD, DH), jnp.float32),
            pltpu.SemaphoreType.DMA,
            pltpu.SemaphoreType.DMA,
            pltpu.SemaphoreType.DMA,
        ],
    )(src3, dst3, u)


def _dinv_block(deg_ref, i, n):
    deg = deg_ref[0, :, 0:1] + deg_ref[1, :, 0:1] + 1.0
    dinv = lax.rsqrt(deg)
    rows = i * BM + lax.broadcasted_iota(jnp.int32, (BM, 1), 0)
    return jnp.where(rows < n, dinv, 0.0)


def _enc_body(x_ref, We1_ref, be1_ref, We2_ref, be2_ref, h_ref):
    h = jnp.maximum(
        jnp.dot(x_ref[...], We1_ref[...], preferred_element_type=jnp.float32)
        + be1_ref[...], 0.0)
    h_ref[...] = (
        jnp.dot(h, We2_ref[...], preferred_element_type=jnp.float32)
        + be2_ref[...])


def _enc_call(x_p, We1, be1, We2, be2):
    grid = (N_PAD // BM,)
    return pl.pallas_call(
        _enc_body,
        grid=grid,
        in_specs=[
            pl.BlockSpec((BM, DH), lambda i: (i, 0)),
            pl.BlockSpec((DH, DH), lambda i: (0, 0)),
            pl.BlockSpec((1, DH), lambda i: (0, 0)),
            pl.BlockSpec((DH, DH), lambda i: (0, 0)),
            pl.BlockSpec((1, DH), lambda i: (0, 0)),
        ],
        out_specs=pl.BlockSpec((BM, DH), lambda i: (i, 0)),
        out_shape=jax.ShapeDtypeStruct((N_PAD, DH), jnp.float32),
    )(x_p, We1, be1, We2, be2)


def _u_body(h_ref, deg_ref, W_ref, u_ref, *, n):
    i = pl.program_id(0)
    dinv = _dinv_block(deg_ref, i, n)
    u_ref[...] = jnp.dot(h_ref[...], W_ref[...],
                         preferred_element_type=jnp.float32) * dinv


def _u_call(h, deg, W, n):
    grid = (N_PAD // BM,)
    return pl.pallas_call(
        functools.partial(_u_body, n=n),
        grid=grid,
        in_specs=[
            pl.BlockSpec((BM, DH), lambda i: (i, 0)),
            pl.BlockSpec((NC, BM, DEGW), lambda i: (0, i, 0)),
            pl.BlockSpec((DH, DH), lambda i: (0, 0)),
        ],
        out_specs=pl.BlockSpec((BM, DH), lambda i: (i, 0)),
        out_shape=jax.ShapeDtypeStruct((N_PAD, DH), jnp.float32),
    )(h, deg, W)


def _mid_body(s_ref, u_ref, deg_ref, b_ref, W_ref, out_ref, *, n):
    i = pl.program_id(0)
    dinv = _dinv_block(deg_ref, i, n)
    h = jnp.maximum(
        dinv * (s_ref[0] + s_ref[1] + u_ref[...]) + b_ref[...], 0.0)
    out_ref[...] = jnp.dot(h, W_ref[...],
                           preferred_element_type=jnp.float32) * dinv


def _mid_call(sacc, u, deg, b, W, n):
    grid = (N_PAD // BM,)
    return pl.pallas_call(
        functools.partial(_mid_body, n=n),
        grid=grid,
        in_specs=[
            pl.BlockSpec((NC, BM, DH), lambda i: (0, i, 0)),
            pl.BlockSpec((BM, DH), lambda i: (i, 0)),
            pl.BlockSpec((NC, BM, DEGW), lambda i: (0, i, 0)),
            pl.BlockSpec((1, DH), lambda i: (0, 0)),
            pl.BlockSpec((DH, DH), lambda i: (0, 0)),
        ],
        out_specs=pl.BlockSpec((BM, DH), lambda i: (i, 0)),
        out_shape=jax.ShapeDtypeStruct((N_PAD, DH), jnp.float32),
    )(sacc, u, deg, b, W)


def _fin_body(s_ref, u_ref, deg_ref, b_ref, batch_ref, Wd1_ref, bd1_ref,
              Wd2_ref, bd2_ref, out_ref, g_acc, *, n):
    i = pl.program_id(0)
    dinv = _dinv_block(deg_ref, i, n)
    h = jnp.maximum(
        dinv * (s_ref[0] + s_ref[1] + u_ref[...]) + b_ref[...], 0.0)
    onehot = (batch_ref[...] ==
              lax.broadcasted_iota(jnp.int32, (NUM_G, 1), 0)
              ).astype(jnp.float32)
    contrib = jnp.dot(onehot, h, preferred_element_type=jnp.float32)

    @pl.when(i == 0)
    def _():
        g_acc[...] = contrib

    @pl.when(i > 0)
    def _():
        g_acc[...] = g_acc[...] + contrib

    @pl.when(i == pl.num_programs(0) - 1)
    def _():
        g = jnp.maximum(
            jnp.dot(g_acc[...], Wd1_ref[...],
                    preferred_element_type=jnp.float32) + bd1_ref[...], 0.0)
        out_ref[...] = (
            jnp.dot(g, Wd2_ref[...], preferred_element_type=jnp.float32)
            + bd2_ref[...])


def _fin_call(sacc, u, deg, b, batch_p, Wd1, bd1, Wd2, bd2, n, d_out):
    grid = (N_PAD // BM,)
    return pl.pallas_call(
        functools.partial(_fin_body, n=n),
        grid=grid,
        in_specs=[
            pl.BlockSpec((NC, BM, DH), lambda i: (0, i, 0)),
            pl.BlockSpec((BM, DH), lambda i: (i, 0)),
            pl.BlockSpec((NC, BM, DEGW), lambda i: (0, i, 0)),
            pl.BlockSpec((1, DH), lambda i: (0, 0)),
            pl.BlockSpec((1, BM), lambda i: (0, i)),
            pl.BlockSpec((DH, DH), lambda i: (0, 0)),
            pl.BlockSpec((1, DH), lambda i: (0, 0)),
            pl.BlockSpec((DH, d_out), lambda i: (0, 0)),
            pl.BlockSpec((1, d_out), lambda i: (0, 0)),
        ],
        out_specs=pl.BlockSpec((NUM_G, d_out), lambda i: (0, 0)),
        out_shape=jax.ShapeDtypeStruct((NUM_G, d_out), jnp.float32),
        scratch_shapes=[pltpu.VMEM((NUM_G, DH), jnp.float32)],
    )(sacc, u, deg, b, batch_p, Wd1, bd1, Wd2, bd2)


def kernel(x, edge_index, batch, We1, be1, We2, be2, Wc1, bc1, Wc2, bc2,
           Wc3, bc3, Wd1, bd1, Wd2, bd2):
    n, _ = x.shape
    e = edge_index.shape[1]
    d_out = Wd2.shape[1]

    quantum = NC * NS * CHUNK * DEG_GRP
    ep = ((e + quantum - 1) // quantum) * quantum
    pad_e = ep - e
    n_chunks = ep // (NC * NS * CHUNK)
    assert n_chunks % DEG_GRP == 0
    src = jnp.concatenate(
        [edge_index[0], jnp.full((pad_e,), n, jnp.int32)])
    dst = jnp.concatenate(
        [edge_index[1], jnp.full((pad_e,), n, jnp.int32)])
    dst3 = dst.reshape(NC * NS, n_chunks, CHUNK)

    x_p = jnp.pad(x, ((0, N_PAD - n), (0, 0)))
    batch_p = jnp.pad(batch, (0, N_PAD - n),
                      constant_values=NUM_G).reshape(1, N_PAD)

    be1_r, be2_r = be1.reshape(1, -1), be2.reshape(1, -1)
    bc1_r, bc2_r, bc3_r = (b.reshape(1, -1) for b in (bc1, bc2, bc3))
    bd1_r, bd2_r = bd1.reshape(1, -1), bd2.reshape(1, -1)

    deg = _deg_call(dst3)
    h = _enc_call(x_p, We1, be1_r, We2, be2_r)

    u = _u_call(h, deg, Wc1, n)
    s = _edge_call(src, dst, u)
    u = _mid_call(s, u, deg, bc1_r, Wc2, n)
    s = _edge_call(src, dst, u)
    u = _mid_call(s, u, deg, bc2_r, Wc3, n)
    s = _edge_call(src, dst, u)
    return _fin_call(s, u, deg, bc3_r, batch_p, Wd1, bd1_r, Wd2, bd2_r,
                     n, d_out)


# speedup vs baseline: 2.4067x; 2.4067x over previous
import functools

import jax
import jax.numpy as jnp
from jax import lax
from jax.experimental import pallas as pl
from jax.experimental.pallas import tpu as pltpu
from jax.experimental.pallas import tpu_sc as plsc

NC = 2
NS = 16
N_PAD = 10240
CHUNK = 128
DH = 128
DEGW = 16
NUM_G = 64
BM = 1280
FRAC0 = 0.73


def _mesh():
    return plsc.VectorSubcoreMesh(core_axis_name="c", subcore_axis_name="s")


DEG_GRP = 8


def _deg_body(dst_hbm, out_hbm, dst_all, ones_v, acc, sem):
    c = lax.axis_index("c")
    s = lax.axis_index("s")
    wid = c * NS + s
    n_chunks = dst_hbm.shape[1]
    rows_per_tile = N_PAD // NS

    def fill(i, _):
        ones_v[i, :] = jnp.full((DEGW,), 1.0, jnp.float32)
        ones_v[CHUNK + i, :] = jnp.zeros((DEGW,), jnp.float32)
        return 0

    lax.fori_loop(0, CHUNK, fill, 0)

    pltpu.sync_copy(dst_hbm.at[wid], dst_all)

    def zero(j, _):
        pltpu.sync_copy(ones_v.at[pl.ds(CHUNK, CHUNK)],
                        acc.at[pl.ds(s * rows_per_tile + j * CHUNK, CHUNK)])
        return 0

    lax.fori_loop(0, rows_per_tile // CHUNK, zero, 0)
    plsc.subcore_barrier()

    ones = ones_v.at[pl.ds(0, CHUNK)]

    def group(g, _):
        for b in range(DEG_GRP):
            j = g * DEG_GRP + b
            pltpu.async_copy(ones, acc.at[dst_all.at[j]], sem, add=True)
        for b in range(DEG_GRP):
            pltpu.make_async_copy(ones, acc.at[dst_all.at[0]], sem).wait()
        return 0

    lax.fori_loop(0, n_chunks // DEG_GRP, group, 0)
    plsc.subcore_barrier()

    pltpu.sync_copy(acc.at[pl.ds(s * rows_per_tile, rows_per_tile)],
                    out_hbm.at[c, pl.ds(s * rows_per_tile, rows_per_tile)])


def _deg_call(dst3):
    n_chunks = dst3.shape[1]
    return pl.kernel(
        _deg_body,
        out_type=jax.ShapeDtypeStruct((NC, N_PAD, DEGW), jnp.float32),
        mesh=_mesh(),
        scratch_types=[
            pltpu.VMEM((n_chunks, CHUNK), jnp.int32),
            pltpu.VMEM((2 * CHUNK, DEGW), jnp.float32),
            pltpu.VMEM_SHARED((N_PAD, DEGW), jnp.float32),
            pltpu.SemaphoreType.DMA,
        ],
    )(dst3)


def _edge_body(src_hbm, dst_hbm, u_hbm, out_hbm, src_v, dst_v, rows,
               acc, isem, gsem, ssem, *, t0, t1):
    c = lax.axis_index("c")
    s = lax.axis_index("s")
    n_chunks = jnp.where(c == 0, t0, t1)
    tile_base = jnp.where(c == 0, s * t0, NS * t0 + s * t1) * CHUNK
    rows_per_tile = N_PAD // NS

    def fill(i, _):
        def fill_lane(k, _):
            rows[0, i, pl.ds(k * 16, 16)] = jnp.zeros((16,), jnp.float32)
            return 0
        lax.fori_loop(0, DH // 16, fill_lane, 0)
        return 0

    lax.fori_loop(0, CHUNK, fill, 0)

    def zero(j, _):
        pltpu.sync_copy(rows.at[0],
                        acc.at[pl.ds(s * rows_per_tile + j * CHUNK, CHUNK)])
        return 0

    lax.fori_loop(0, rows_per_tile // CHUNK, zero, 0)
    plsc.subcore_barrier()

    def idx_load(j, slot):
        base = pl.multiple_of(tile_base + j * CHUNK, CHUNK)
        pltpu.async_copy(src_hbm.at[pl.ds(base, CHUNK)], src_v.at[slot],
                         isem)
        pltpu.async_copy(dst_hbm.at[pl.ds(base, CHUNK)], dst_v.at[slot],
                         isem)

    def idx_wait():
        pltpu.make_async_copy(src_hbm.at[pl.ds(0, CHUNK)], src_v.at[0],
                              isem).wait()
        pltpu.make_async_copy(dst_hbm.at[pl.ds(0, CHUNK)], dst_v.at[0],
                              isem).wait()

    def gather(buf, slot):
        pltpu.async_copy(u_hbm.at[src_v.at[slot]], rows.at[buf], gsem)

    def gather_wait(buf):
        pltpu.make_async_copy(u_hbm.at[src_v.at[0]], rows.at[buf],
                              gsem).wait()

    def scatter(buf, slot):
        pltpu.async_copy(rows.at[buf], acc.at[dst_v.at[slot]], ssem,
                         add=True)

    def scatter_wait(buf):
        pltpu.make_async_copy(rows.at[buf], acc.at[dst_v.at[0]],
                              ssem).wait()

    idx_load(0, 0)
    idx_load(1, 1)
    idx_wait()
    gather(0, 0)

    def step(j, _):
        p = j % 2
        q = 1 - p
        gather_wait(p)
        scatter(p, j % 3)

        @pl.when(j + 1 < n_chunks)
        def _():
            @pl.when(j >= 1)
            def _():
                scatter_wait(q)

            @pl.when(j + 2 < n_chunks)
            def _():
                idx_load(j + 2, (j + 2) % 3)

            idx_wait()
            gather(q, (j + 1) % 3)
        return 0

    lax.fori_loop(0, n_chunks, step, 0)
    scatter_wait((n_chunks - 1) % 2)
    plsc.subcore_barrier()

    pltpu.sync_copy(acc.at[pl.ds(s * rows_per_tile, rows_per_tile)],
                    out_hbm.at[c, pl.ds(s * rows_per_tile, rows_per_tile)])


def _edge_call(src, dst, u, t0, t1):
    return pl.kernel(
        functools.partial(_edge_body, t0=t0, t1=t1),
        out_type=jax.ShapeDtypeStruct((NC, N_PAD, DH), jnp.float32),
        mesh=_mesh(),
        scratch_types=[
            pltpu.VMEM((3, CHUNK), jnp.int32),
            pltpu.VMEM((3, CHUNK), jnp.int32),
            pltpu.VMEM((2, CHUNK, DH), jnp.float32),
            pltpu.VMEM_SHARED((N_PAD, DH), jnp.float32),
            pltpu.SemaphoreType.DMA,
            pltpu.SemaphoreType.DMA,
            pltpu.SemaphoreType.DMA,
        ],
    )(src, dst, u)


def _dinv_block(deg_ref, i, n):
    deg = deg_ref[0, :, 0:1] + deg_ref[1, :, 0:1] + 1.0
    dinv = lax.rsqrt(deg)
    rows = i * BM + lax.broadcasted_iota(jnp.int32, (BM, 1), 0)
    return jnp.where(rows < n, dinv, 0.0)


def _enc_body(x_ref, We1_ref, be1_ref, We2_ref, be2_ref, h_ref):
    h = jnp.maximum(
        jnp.dot(x_ref[...], We1_ref[...], preferred_element_type=jnp.float32)
        + be1_ref[...], 0.0)
    h_ref[...] = (
        jnp.dot(h, We2_ref[...], preferred_element_type=jnp.float32)
        + be2_ref[...])


def _enc_call(x_p, We1, be1, We2, be2):
    grid = (N_PAD // BM,)
    return pl.pallas_call(
        _enc_body,
        grid=grid,
        in_specs=[
            pl.BlockSpec((BM, DH), lambda i: (i, 0)),
            pl.BlockSpec((DH, DH), lambda i: (0, 0)),
            pl.BlockSpec((1, DH), lambda i: (0, 0)),
            pl.BlockSpec((DH, DH), lambda i: (0, 0)),
            pl.BlockSpec((1, DH), lambda i: (0, 0)),
        ],
        out_specs=pl.BlockSpec((BM, DH), lambda i: (i, 0)),
        out_shape=jax.ShapeDtypeStruct((N_PAD, DH), jnp.float32),
    )(x_p, We1, be1, We2, be2)


def _u_body(h_ref, deg_ref, W_ref, u_ref, *, n):
    i = pl.program_id(0)
    dinv = _dinv_block(deg_ref, i, n)
    u_ref[...] = jnp.dot(h_ref[...], W_ref[...],
                         preferred_element_type=jnp.float32) * dinv


def _u_call(h, deg, W, n):
    grid = (N_PAD // BM,)
    return pl.pallas_call(
        functools.partial(_u_body, n=n),
        grid=grid,
        in_specs=[
            pl.BlockSpec((BM, DH), lambda i: (i, 0)),
            pl.BlockSpec((NC, BM, DEGW), lambda i: (0, i, 0)),
            pl.BlockSpec((DH, DH), lambda i: (0, 0)),
        ],
        out_specs=pl.BlockSpec((BM, DH), lambda i: (i, 0)),
        out_shape=jax.ShapeDtypeStruct((N_PAD, DH), jnp.float32),
    )(h, deg, W)


def _mid_body(s_ref, u_ref, deg_ref, b_ref, W_ref, out_ref, *, n):
    i = pl.program_id(0)
    dinv = _dinv_block(deg_ref, i, n)
    h = jnp.maximum(
        dinv * (s_ref[0] + s_ref[1] + u_ref[...]) + b_ref[...], 0.0)
    out_ref[...] = jnp.dot(h, W_ref[...],
                           preferred_element_type=jnp.float32) * dinv


def _mid_call(sacc, u, deg, b, W, n):
    grid = (N_PAD // BM,)
    return pl.pallas_call(
        functools.partial(_mid_body, n=n),
        grid=grid,
        in_specs=[
            pl.BlockSpec((NC, BM, DH), lambda i: (0, i, 0)),
            pl.BlockSpec((BM, DH), lambda i: (i, 0)),
            pl.BlockSpec((NC, BM, DEGW), lambda i: (0, i, 0)),
            pl.BlockSpec((1, DH), lambda i: (0, 0)),
            pl.BlockSpec((DH, DH), lambda i: (0, 0)),
        ],
        out_specs=pl.BlockSpec((BM, DH), lambda i: (i, 0)),
        out_shape=jax.ShapeDtypeStruct((N_PAD, DH), jnp.float32),
    )(sacc, u, deg, b, W)


def _fin_body(s_ref, u_ref, deg_ref, b_ref, batch_ref, Wd1_ref, bd1_ref,
              Wd2_ref, bd2_ref, out_ref, g_acc, *, n):
    i = pl.program_id(0)
    dinv = _dinv_block(deg_ref, i, n)
    h = jnp.maximum(
        dinv * (s_ref[0] + s_ref[1] + u_ref[...]) + b_ref[...], 0.0)
    onehot = (batch_ref[...] ==
              lax.broadcasted_iota(jnp.int32, (NUM_G, 1), 0)
              ).astype(jnp.float32)
    contrib = jnp.dot(onehot, h, preferred_element_type=jnp.float32)

    @pl.when(i == 0)
    def _():
        g_acc[...] = contrib

    @pl.when(i > 0)
    def _():
        g_acc[...] = g_acc[...] + contrib

    @pl.when(i == pl.num_programs(0) - 1)
    def _():
        g = jnp.maximum(
            jnp.dot(g_acc[...], Wd1_ref[...],
                    preferred_element_type=jnp.float32) + bd1_ref[...], 0.0)
        out_ref[...] = (
            jnp.dot(g, Wd2_ref[...], preferred_element_type=jnp.float32)
            + bd2_ref[...])


def _fin_call(sacc, u, deg, b, batch_p, Wd1, bd1, Wd2, bd2, n, d_out):
    grid = (N_PAD // BM,)
    return pl.pallas_call(
        functools.partial(_fin_body, n=n),
        grid=grid,
        in_specs=[
            pl.BlockSpec((NC, BM, DH), lambda i: (0, i, 0)),
            pl.BlockSpec((BM, DH), lambda i: (i, 0)),
            pl.BlockSpec((NC, BM, DEGW), lambda i: (0, i, 0)),
            pl.BlockSpec((1, DH), lambda i: (0, 0)),
            pl.BlockSpec((1, BM), lambda i: (0, i)),
            pl.BlockSpec((DH, DH), lambda i: (0, 0)),
            pl.BlockSpec((1, DH), lambda i: (0, 0)),
            pl.BlockSpec((DH, d_out), lambda i: (0, 0)),
            pl.BlockSpec((1, d_out), lambda i: (0, 0)),
        ],
        out_specs=pl.BlockSpec((NUM_G, d_out), lambda i: (0, 0)),
        out_shape=jax.ShapeDtypeStruct((NUM_G, d_out), jnp.float32),
        scratch_shapes=[pltpu.VMEM((NUM_G, DH), jnp.float32)],
    )(sacc, u, deg, b, batch_p, Wd1, bd1, Wd2, bd2)


def kernel(x, edge_index, batch, We1, be1, We2, be2, Wc1, bc1, Wc2, bc2,
           Wc3, bc3, Wd1, bd1, Wd2, bd2):
    n, _ = x.shape
    e = edge_index.shape[1]
    d_out = Wd2.shape[1]

    quantum = NS * CHUNK
    ep = ((e + quantum - 1) // quantum) * quantum
    t_total = ep // quantum
    t0 = max(2, min(t_total - 2, round(FRAC0 * t_total)))
    t1 = t_total - t0
    src = jnp.concatenate(
        [edge_index[0], jnp.full((ep - e,), n, jnp.int32)])
    dst = jnp.concatenate(
        [edge_index[1], jnp.full((ep - e,), n, jnp.int32)])
    quantum_d = NC * NS * CHUNK * DEG_GRP
    ep_d = ((e + quantum_d - 1) // quantum_d) * quantum_d
    dst3 = jnp.concatenate(
        [edge_index[1], jnp.full((ep_d - e,), n, jnp.int32)]
    ).reshape(NC * NS, ep_d // (NC * NS * CHUNK), CHUNK)

    x_p = jnp.pad(x, ((0, N_PAD - n), (0, 0)))
    batch_p = jnp.pad(batch, (0, N_PAD - n),
                      constant_values=NUM_G).reshape(1, N_PAD)

    be1_r, be2_r = be1.reshape(1, -1), be2.reshape(1, -1)
    bc1_r, bc2_r, bc3_r = (b.reshape(1, -1) for b in (bc1, bc2, bc3))
    bd1_r, bd2_r = bd1.reshape(1, -1), bd2.reshape(1, -1)

    deg = _deg_call(dst3)
    h = _enc_call(x_p, We1, be1_r, We2, be2_r)

    u = _u_call(h, deg, Wc1, n)
    s = _edge_call(src, dst, u, t0, t1)
    u = _mid_call(s, u, deg, bc1_r, Wc2, n)
    s = _edge_call(src, dst, u, t0, t1)
    u = _mid_call(s, u, deg, bc2_r, Wc3, n)
    s = _edge_call(src, dst, u, t0, t1)
    return _fin_call(s, u, deg, bc3_r, batch_p, Wd1, bd1_r, Wd2, bd2_r,
                     n, d_out)
